# baseline (speedup 1.0000x reference)
import functools

import jax
import jax.numpy as jnp
from jax.experimental import pallas as pl
from jax.experimental.pallas import tpu as pltpu

_EPAD = 128


def _body(E, sp_ref, x_ref, w1_ref, b1_ref, w2p_ref, b2p_ref, tab_ref,
          out_ref, log_ref):
    j = pl.program_id(0)
    nj = pl.num_programs(0)

    pre = jnp.dot(x_ref[...].astype(jnp.bfloat16),
                  w1_ref[...].astype(jnp.bfloat16),
                  preferred_element_type=jnp.float32)
    h = jax.nn.gelu(pre + b1_ref[...])
    plog = jnp.dot(h.astype(jnp.bfloat16),
                   w2p_ref[...].astype(jnp.bfloat16),
                   preferred_element_type=jnp.float32)

    @pl.when(j == 0)
    def _():
        log_ref[...] = plog + b2p_ref[...]

    @pl.when(j != 0)
    def _():
        log_ref[...] = log_ref[...] + plog

    @pl.when(j == nj - 1)
    def _():
        lg = log_ref[...]
        m = jnp.max(lg)
        rows = jax.lax.broadcasted_iota(jnp.int32, lg.shape, 0)
        cols = jax.lax.broadcasted_iota(jnp.int32, lg.shape, 1)
        flat = rows * E + cols
        idx = jnp.min(jnp.where(lg == m, flat, jnp.int32(2**30)))
        row2 = tab_ref[0, pl.ds(idx // 2, 1), :]
        half = row2.shape[-1] // 2
        out_ref[...] = jnp.where((idx % 2) == 0, row2[:, :half], row2[:, half:])


def kernel(predicate, W1, b1, W2, b2, expert_tables, input):
    T, D = predicate.shape
    H = W1.shape[1]
    E = W2.shape[1]
    n_tab, ROWS, ED = expert_tables.shape
    tab2 = expert_tables.reshape(n_tab, ROWS // 2, 2 * ED)

    HB = 256
    J = H // HB

    W2p = jnp.zeros((H, _EPAD), jnp.float32).at[:, :E].set(W2)
    b2p = jnp.full((1, _EPAD), -1e30, jnp.float32).at[0, :E].set(b2)
    b1r = b1.reshape(1, H)
    sp = jnp.asarray(input, jnp.int32).reshape(1)

    grid_spec = pltpu.PrefetchScalarGridSpec(
        num_scalar_prefetch=1,
        grid=(J,),
        in_specs=[
            pl.BlockSpec((T, D), lambda j, sp: (0, 0)),
            pl.BlockSpec((D, HB), lambda j, sp: (0, j)),
            pl.BlockSpec((1, HB), lambda j, sp: (0, j)),
            pl.BlockSpec((HB, _EPAD), lambda j, sp: (j, 0)),
            pl.BlockSpec((1, _EPAD), lambda j, sp: (0, 0)),
            pl.BlockSpec((1, ROWS // 2, 2 * ED), lambda j, sp: (sp[0], 0, 0)),
        ],
        out_specs=pl.BlockSpec((1, ED), lambda j, sp: (0, 0)),
        scratch_shapes=[
            pltpu.VMEM((T, _EPAD), jnp.float32),
        ],
    )

    out = pl.pallas_call(
        functools.partial(_body, E),
        grid_spec=grid_spec,
        out_shape=jax.ShapeDtypeStruct((1, ED), jnp.float32),
        compiler_params=pltpu.CompilerParams(
            dimension_semantics=("arbitrary",),
        ),
    )(sp, predicate, W1, b1r, W2p, b2p, tab2)
    return out.reshape(ED)
